# Initial kernel scaffold; baseline (speedup 1.0000x reference)
import jax
import jax.numpy as jnp
from jax.experimental import pallas as pl


def kernel(user_emb, item_emb, gating_w, gating_b, sgating_w, sgating_b, att_mat, att_agg, hs_rows, hs_cols, hs_vals, hj_rows, hj_cols, hj_vals, hp_rows, hp_cols, hp_vals, inter_rows, inter_cols, inter_vals):
    raise NotImplementedError("write your pallas kernel here")


# speedup vs baseline: 2.5164x; 2.5164x over previous
import functools

import jax
import jax.numpy as jnp
from jax import lax
from jax.experimental import pallas as pl
from jax.experimental.pallas import tpu as pltpu
from jax.experimental.pallas import tpu_sc as plsc

N = 50000
D = 64
H = D // 2
E = 800000
C = 128
NSUB = 16
CH_TOTAL = E // C
CH_BASE = CH_TOTAL // NSUB
CH_REM = CH_TOTAL % NSUB
ROWS_PER_TILE = N // NSUB
WB = 625
RB = 2000
GRID = N // RB

_f32 = jnp.float32


def _spmm_body(x2, rows, cols, vals, out, rvm, cvm, ivm, vvm, gbuf, wbuf,
               acc, sem):
    c = lax.axis_index("c")
    s = lax.axis_index("s")

    zeros16 = jnp.zeros((16,), _f32)

    def _zero_wbuf(i, carry):
        wbuf[i, 0:16] = zeros16
        wbuf[i, 16:32] = zeros16
        return carry

    lax.fori_loop(0, WB, _zero_wbuf, 0)
    for k in range(N // NSUB // WB):
        pltpu.sync_copy(wbuf, acc.at[pl.ds(s * ROWS_PER_TILE + k * WB, WB), :])
    plsc.subcore_barrier()

    cs = CH_BASE * s + jnp.minimum(s, CH_REM)
    nch = CH_BASE + (s < CH_REM).astype(jnp.int32)

    def _chunk(ch, carry):
        eb = (cs + ch) * C
        pltpu.sync_copy(rows.at[pl.ds(eb, C)], rvm)
        pltpu.sync_copy(cols.at[pl.ds(eb, C)], cvm)
        pltpu.sync_copy(vals.at[pl.ds(eb, C)], vvm)

        def _xform(j, carry2):
            t = cvm[pl.ds(j * 16, 16)]
            ivm[pl.ds(j * 16, 16)] = t * 2 + c
            return carry2

        lax.fori_loop(0, C // 16, _xform, 0)
        pltpu.async_copy(x2.at[ivm], gbuf, sem).wait()

        def _scale(g, carry2):
            v16 = vvm[pl.ds(g * 16, 16)]
            e0 = g * 16
            for j in range(16):
                v = v16[j]
                gbuf[e0 + j, 0:16] = gbuf[e0 + j, 0:16] * v
                gbuf[e0 + j, 16:32] = gbuf[e0 + j, 16:32] * v
            return carry2

        lax.fori_loop(0, C // 16, _scale, 0)
        pltpu.sync_copy(gbuf, acc.at[rvm], add=True)
        return carry

    lax.fori_loop(0, nch, _chunk, 0)
    plsc.subcore_barrier()

    for k in range(N // NSUB // WB):
        r0 = s * ROWS_PER_TILE + k * WB
        pltpu.sync_copy(acc.at[pl.ds(r0, WB), :], wbuf)
        pltpu.sync_copy(wbuf, out.at[pl.ds(r0, WB), c, :])


@jax.jit
def _spmm(x, rows, cols, vals):
    mesh = plsc.VectorSubcoreMesh(core_axis_name="c", subcore_axis_name="s")
    x2 = x.reshape(2 * N, H)
    out = pl.kernel(
        _spmm_body,
        out_type=jax.ShapeDtypeStruct((N, 2, H), _f32),
        mesh=mesh,
        compiler_params=pltpu.CompilerParams(use_tc_tiling_on_sc=False),
        scratch_types=[
            pltpu.VMEM((C,), jnp.int32),
            pltpu.VMEM((C,), jnp.int32),
            pltpu.VMEM((C,), jnp.int32),
            pltpu.VMEM((C,), _f32),
            pltpu.VMEM((C, H), _f32),
            pltpu.VMEM((WB, H), _f32),
            pltpu.VMEM_SHARED((N, H), _f32),
            pltpu.SemaphoreType.DMA,
        ],
    )(x2, rows, cols, vals)
    return out.reshape(N, D)


def _gate_k(em_ref, w_ref, b_ref, o0, o1, o2, o3):
    em = em_ref[...]
    for i, o in enumerate((o0, o1, o2, o3)):
        z = jnp.dot(em, w_ref[i], preferred_element_type=_f32) + b_ref[i][None, :]
        o[...] = em * jax.nn.sigmoid(z)


def _attn_mix(u0, u1, u2, am_ref, aa_ref):
    ws = []
    for u in (u0, u1, u2):
        t = jnp.dot(u, am_ref[...], preferred_element_type=_f32)
        ws.append(jnp.sum(aa_ref[...] * t, axis=1))
    m = jnp.maximum(jnp.maximum(ws[0], ws[1]), ws[2])
    es = [jnp.exp(w - m) for w in ws]
    tot = es[0] + es[1] + es[2]
    mixed = es[0][:, None] * u0 + es[1][:, None] * u1 + es[2][:, None] * u2
    return mixed / tot[:, None]


def _mix_k(u0, u1, u2, us, am_ref, aa_ref, mo):
    mixed = _attn_mix(u0[...], u1[...], u2[...], am_ref, aa_ref)
    mo[...] = (mixed + us[...]) * 0.5


def _normacc_k(a0, x0, a1, x1, a2, x2, a3, x3, a4, x4, o0, o1, o2, o3, o4):
    for a, x, o in ((a0, x0, o0), (a1, x1, o1), (a2, x2, o2), (a3, x3, o3),
                    (a4, x4, o4)):
        xx = x[...]
        n = jnp.sqrt(jnp.sum(xx * xx, axis=1, keepdims=True))
        o[...] = a[...] + xx / jnp.maximum(n, 1e-12)


def _final_k(a0, a1, a2, aS, am_ref, aa_ref, sw_ref, sb_ref, fu, s0, s1, s2):
    mixed = _attn_mix(a0[...], a1[...], a2[...], am_ref, aa_ref)
    f = mixed + aS[...] * 0.5
    fu[...] = f
    for i, o in enumerate((s0, s1, s2)):
        z = jnp.dot(f, sw_ref[i], preferred_element_type=_f32) + sb_ref[i][None, :]
        o[...] = f * jax.nn.sigmoid(z)


_row_spec = pl.BlockSpec((RB, D), lambda i: (i, 0))


def _full_spec(shape):
    return pl.BlockSpec(shape, lambda i: tuple(0 for _ in shape))


def _rows_out(n):
    return tuple(jax.ShapeDtypeStruct((N, D), _f32) for _ in range(n))


@jax.jit
def _gate(em, w, b):
    return pl.pallas_call(
        _gate_k,
        grid=(GRID,),
        in_specs=[_row_spec, _full_spec((4, D, D)), _full_spec((4, D))],
        out_specs=(_row_spec,) * 4,
        out_shape=_rows_out(4),
    )(em, w, b)


@jax.jit
def _mix(u0, u1, u2, us, am, aa):
    return pl.pallas_call(
        _mix_k,
        grid=(GRID,),
        in_specs=[_row_spec] * 4 + [_full_spec((D, D)), _full_spec((1, D))],
        out_specs=_row_spec,
        out_shape=jax.ShapeDtypeStruct((N, D), _f32),
    )(u0, u1, u2, us, am, aa)


@jax.jit
def _normacc(a0, x0, a1, x1, a2, x2, a3, x3, a4, x4):
    return pl.pallas_call(
        _normacc_k,
        grid=(GRID,),
        in_specs=[_row_spec] * 10,
        out_specs=(_row_spec,) * 5,
        out_shape=_rows_out(5),
    )(a0, x0, a1, x1, a2, x2, a3, x3, a4, x4)


@jax.jit
def _final(a0, a1, a2, aS, am, aa, sw, sb):
    return pl.pallas_call(
        _final_k,
        grid=(GRID,),
        in_specs=[_row_spec] * 4
        + [_full_spec((D, D)), _full_spec((1, D)), _full_spec((4, D, D)),
           _full_spec((4, D))],
        out_specs=(_row_spec,) * 4,
        out_shape=_rows_out(4),
    )(a0, a1, a2, aS, am, aa, sw, sb)


def kernel(user_emb, item_emb, gating_w, gating_b, sgating_w, sgating_b,
           att_mat, att_agg, hs_rows, hs_cols, hs_vals, hj_rows, hj_cols,
           hj_vals, hp_rows, hp_cols, hp_vals, inter_rows, inter_cols,
           inter_vals):
    i32 = jnp.int32
    hs_rows, hs_cols = hs_rows.astype(i32), hs_cols.astype(i32)
    hj_rows, hj_cols = hj_rows.astype(i32), hj_cols.astype(i32)
    hp_rows, hp_cols = hp_rows.astype(i32), hp_cols.astype(i32)
    inter_rows, inter_cols = inter_rows.astype(i32), inter_cols.astype(i32)

    u0, u1, u2, us = _gate(user_emb, gating_w, gating_b)
    acc0, acc1, acc2, accS, accI = u0, u1, u2, us, item_emb
    it = item_emb
    for _ in range(2):
        mixed = _mix(u0, u1, u2, us, att_mat, att_agg)
        u0n = _spmm(u0, hs_rows, hs_cols, hs_vals)
        u1n = _spmm(u1, hj_rows, hj_cols, hj_vals)
        u2n = _spmm(u2, hp_rows, hp_cols, hp_vals)
        itn = _spmm(mixed, inter_cols, inter_rows, inter_vals)
        usn = _spmm(it, inter_rows, inter_cols, inter_vals)
        acc0, acc1, acc2, accS, accI = _normacc(
            acc0, u0n, acc1, u1n, acc2, u2n, accS, usn, accI, itn)
        u0, u1, u2, us, it = u0n, u1n, u2n, usn, itn
    fu, s0, s1, s2 = _final(acc0, acc1, acc2, accS, att_mat, att_agg,
                            sgating_w, sgating_b)
    return (fu, accI, (s0, s1, s2))


# speedup vs baseline: 4.5699x; 1.8160x over previous
import functools

import jax
import jax.numpy as jnp
from jax import lax
from jax.experimental import pallas as pl
from jax.experimental.pallas import tpu as pltpu
from jax.experimental.pallas import tpu_sc as plsc

N = 50000
D = 64
H = D // 2
E = 800000
C = 80
NSUB = 16
MB = 2000
NBLK = E // NSUB // MB
NCH = MB // C
ROWS_PER_TILE = N // NSUB
WB = 125
RB = 2000
GRID = N // RB

_f32 = jnp.float32


def _spmm_body(x2, rows, cols, vals, out, rvm, cvm, vvm, ivm, rix, gbuf0,
               gbuf1, wbuf, acc, gsem, ssem):
    c = lax.axis_index("c")
    s = lax.axis_index("s")

    zeros16 = jnp.zeros((16,), _f32)

    def _zero_wbuf(i, carry):
        wbuf[i, 0:16] = zeros16
        wbuf[i, 16:32] = zeros16
        return carry

    lax.fori_loop(0, WB, _zero_wbuf, 0)

    def _zero_acc(k, carry):
        pltpu.sync_copy(wbuf, acc.at[pl.ds(s * ROWS_PER_TILE + k * WB, WB), :])
        return carry

    lax.fori_loop(0, N // NSUB // WB, _zero_acc, 0)
    plsc.subcore_barrier()

    e_base = s * (E // NSUB)

    def _scale(buf, ch):
        def _grp(g, carry):
            b = ch * C + g * 16
            v16 = vvm[pl.ds(b, 16)]
            for j in range(16):
                r = g * 16 + j
                buf[r, 0:16] = buf[r, 0:16] * v16[j]
                buf[r, 16:32] = buf[r, 16:32] * v16[j]
            return carry

        lax.fori_loop(0, C // 16, _grp, 0)

    def _block(blk, carry):
        eb = e_base + blk * MB
        pltpu.sync_copy(rows.at[pl.ds(eb, MB)], rvm)
        pltpu.sync_copy(cols.at[pl.ds(eb, MB)], cvm)
        pltpu.sync_copy(vals.at[pl.ds(eb, MB)], vvm)

        def _xform(k, carry2):
            for g in range(NCH * C // 16 // NCH):
                b = k * C + g * 16
                t = cvm[pl.ds(b, 16)]
                ivm[k, pl.ds(g * 16, 16)] = t * 2 + c
                rix[k, pl.ds(g * 16, 16)] = rvm[pl.ds(b, 16)]
            return carry2

        lax.fori_loop(0, NCH, _xform, 0)

        pltpu.async_copy(x2.at[ivm.at[0]], gbuf0, gsem)

        def _chunk(i, carry2):
            def _process(buf, other):
                pltpu.make_async_copy(x2.at[ivm.at[i]], buf, gsem).wait()

                @pl.when(i >= 1)
                def _():
                    pltpu.make_async_copy(
                        other, acc.at[rix.at[0]], ssem).wait()

                @pl.when(i + 1 < NCH)
                def _():
                    pltpu.async_copy(x2.at[ivm.at[i + 1]], other, gsem)

                _scale(buf, i)
                pltpu.async_copy(buf, acc.at[rix.at[i]], ssem, add=True)

            @pl.when(lax.rem(i, 2) == 0)
            def _():
                _process(gbuf0, gbuf1)

            @pl.when(lax.rem(i, 2) == 1)
            def _():
                _process(gbuf1, gbuf0)

            return carry2

        lax.fori_loop(0, NCH, _chunk, 0)
        pltpu.make_async_copy(gbuf0, acc.at[rix.at[0]], ssem).wait()
        return carry

    lax.fori_loop(0, NBLK, _block, 0)
    plsc.subcore_barrier()

    def _wb(k, carry):
        r0 = s * ROWS_PER_TILE + k * WB
        pltpu.sync_copy(acc.at[pl.ds(r0, WB), :], wbuf)
        pltpu.sync_copy(wbuf, out.at[pl.ds(r0, WB), c, :])
        return carry

    lax.fori_loop(0, N // NSUB // WB, _wb, 0)


@jax.jit
def _spmm(x, rows, cols, vals):
    mesh = plsc.VectorSubcoreMesh(core_axis_name="c", subcore_axis_name="s")
    x2 = x.reshape(2 * N, H)
    out = pl.kernel(
        _spmm_body,
        out_type=jax.ShapeDtypeStruct((N, 2, H), _f32),
        mesh=mesh,
        compiler_params=pltpu.CompilerParams(use_tc_tiling_on_sc=False),
        scratch_types=[
            pltpu.VMEM((MB,), jnp.int32),
            pltpu.VMEM((MB,), jnp.int32),
            pltpu.VMEM((MB,), _f32),
            pltpu.VMEM((NCH, C), jnp.int32),
            pltpu.VMEM((NCH, C), jnp.int32),
            pltpu.VMEM((C, H), _f32),
            pltpu.VMEM((C, H), _f32),
            pltpu.VMEM((WB, H), _f32),
            pltpu.VMEM_SHARED((N, H), _f32),
            pltpu.SemaphoreType.DMA,
            pltpu.SemaphoreType.DMA,
        ],
    )(x2, rows, cols, vals)
    return out.reshape(N, D)


def _gate_k(em_ref, w_ref, b_ref, o0, o1, o2, o3):
    em = em_ref[...]
    for i, o in enumerate((o0, o1, o2, o3)):
        z = jnp.dot(em, w_ref[i], preferred_element_type=_f32) + b_ref[i][None, :]
        o[...] = em * jax.nn.sigmoid(z)


def _attn_mix(u0, u1, u2, am_ref, aa_ref):
    ws = []
    for u in (u0, u1, u2):
        t = jnp.dot(u, am_ref[...], preferred_element_type=_f32)
        ws.append(jnp.sum(aa_ref[...] * t, axis=1))
    m = jnp.maximum(jnp.maximum(ws[0], ws[1]), ws[2])
    es = [jnp.exp(w - m) for w in ws]
    tot = es[0] + es[1] + es[2]
    mixed = es[0][:, None] * u0 + es[1][:, None] * u1 + es[2][:, None] * u2
    return mixed / tot[:, None]


def _mix_k(u0, u1, u2, us, am_ref, aa_ref, mo):
    mixed = _attn_mix(u0[...], u1[...], u2[...], am_ref, aa_ref)
    mo[...] = (mixed + us[...]) * 0.5


def _normacc_k(a0, x0, a1, x1, a2, x2, a3, x3, a4, x4, o0, o1, o2, o3, o4):
    for a, x, o in ((a0, x0, o0), (a1, x1, o1), (a2, x2, o2), (a3, x3, o3),
                    (a4, x4, o4)):
        xx = x[...]
        n = jnp.sqrt(jnp.sum(xx * xx, axis=1, keepdims=True))
        o[...] = a[...] + xx / jnp.maximum(n, 1e-12)


def _final_k(a0, a1, a2, aS, am_ref, aa_ref, sw_ref, sb_ref, fu, s0, s1, s2):
    mixed = _attn_mix(a0[...], a1[...], a2[...], am_ref, aa_ref)
    f = mixed + aS[...] * 0.5
    fu[...] = f
    for i, o in enumerate((s0, s1, s2)):
        z = jnp.dot(f, sw_ref[i], preferred_element_type=_f32) + sb_ref[i][None, :]
        o[...] = f * jax.nn.sigmoid(z)


_row_spec = pl.BlockSpec((RB, D), lambda i: (i, 0))


def _full_spec(shape):
    return pl.BlockSpec(shape, lambda i: tuple(0 for _ in shape))


def _rows_out(n):
    return tuple(jax.ShapeDtypeStruct((N, D), _f32) for _ in range(n))


@jax.jit
def _gate(em, w, b):
    return pl.pallas_call(
        _gate_k,
        grid=(GRID,),
        in_specs=[_row_spec, _full_spec((4, D, D)), _full_spec((4, D))],
        out_specs=(_row_spec,) * 4,
        out_shape=_rows_out(4),
    )(em, w, b)


@jax.jit
def _mix(u0, u1, u2, us, am, aa):
    return pl.pallas_call(
        _mix_k,
        grid=(GRID,),
        in_specs=[_row_spec] * 4 + [_full_spec((D, D)), _full_spec((1, D))],
        out_specs=_row_spec,
        out_shape=jax.ShapeDtypeStruct((N, D), _f32),
    )(u0, u1, u2, us, am, aa)


@jax.jit
def _normacc(a0, x0, a1, x1, a2, x2, a3, x3, a4, x4):
    return pl.pallas_call(
        _normacc_k,
        grid=(GRID,),
        in_specs=[_row_spec] * 10,
        out_specs=(_row_spec,) * 5,
        out_shape=_rows_out(5),
    )(a0, x0, a1, x1, a2, x2, a3, x3, a4, x4)


@jax.jit
def _final(a0, a1, a2, aS, am, aa, sw, sb):
    return pl.pallas_call(
        _final_k,
        grid=(GRID,),
        in_specs=[_row_spec] * 4
        + [_full_spec((D, D)), _full_spec((1, D)), _full_spec((4, D, D)),
           _full_spec((4, D))],
        out_specs=(_row_spec,) * 4,
        out_shape=_rows_out(4),
    )(a0, a1, a2, aS, am, aa, sw, sb)


def kernel(user_emb, item_emb, gating_w, gating_b, sgating_w, sgating_b,
           att_mat, att_agg, hs_rows, hs_cols, hs_vals, hj_rows, hj_cols,
           hj_vals, hp_rows, hp_cols, hp_vals, inter_rows, inter_cols,
           inter_vals):
    i32 = jnp.int32
    hs_rows, hs_cols = hs_rows.astype(i32), hs_cols.astype(i32)
    hj_rows, hj_cols = hj_rows.astype(i32), hj_cols.astype(i32)
    hp_rows, hp_cols = hp_rows.astype(i32), hp_cols.astype(i32)
    inter_rows, inter_cols = inter_rows.astype(i32), inter_cols.astype(i32)

    u0, u1, u2, us = _gate(user_emb, gating_w, gating_b)
    acc0, acc1, acc2, accS, accI = u0, u1, u2, us, item_emb
    it = item_emb
    for _ in range(2):
        mixed = _mix(u0, u1, u2, us, att_mat, att_agg)
        u0n = _spmm(u0, hs_rows, hs_cols, hs_vals)
        u1n = _spmm(u1, hj_rows, hj_cols, hj_vals)
        u2n = _spmm(u2, hp_rows, hp_cols, hp_vals)
        itn = _spmm(mixed, inter_cols, inter_rows, inter_vals)
        usn = _spmm(it, inter_rows, inter_cols, inter_vals)
        acc0, acc1, acc2, accS, accI = _normacc(
            acc0, u0n, acc1, u1n, acc2, u2n, accS, usn, accI, itn)
        u0, u1, u2, us, it = u0n, u1n, u2n, usn, itn
    fu, s0, s1, s2 = _final(acc0, acc1, acc2, accS, att_mat, att_agg,
                            sgating_w, sgating_b)
    return (fu, accI, (s0, s1, s2))


# speedup vs baseline: 7.5744x; 1.6575x over previous
import functools

import jax
import jax.numpy as jnp
from jax import lax
from jax.experimental import pallas as pl
from jax.experimental.pallas import tpu as pltpu
from jax.experimental.pallas import tpu_sc as plsc

N = 50000
D = 64
H = D // 2
E = 800000
C = 80
NSUB = 16
MB = 2000
NBLK = E // NSUB // MB
NCH = MB // C
ROWS_PER_TILE = N // NSUB
WB = 125
RB = 2000
GRID = N // RB

_f32 = jnp.float32


def _spmm_body(x2, rows, cols, vals, out, rvm, cvm, vvm, ivm, rix, gbuf0,
               gbuf1, gbuf2, gbuf3, wbuf, acc, gsem, ssem):
    c = lax.axis_index("c")
    s = lax.axis_index("s")

    zeros16 = jnp.zeros((16,), _f32)

    def _zero_wbuf(i, carry):
        wbuf[i, 0:16] = zeros16
        wbuf[i, 16:32] = zeros16
        return carry

    lax.fori_loop(0, WB, _zero_wbuf, 0)

    def _zero_acc(k, carry):
        pltpu.sync_copy(wbuf, acc.at[pl.ds(s * ROWS_PER_TILE + k * WB, WB), :])
        return carry

    lax.fori_loop(0, N // NSUB // WB, _zero_acc, 0)
    plsc.subcore_barrier()

    e_base = s * (E // NSUB)

    def _scale(buf, ch):
        for g in range(C // 16):
            v16 = vvm[pl.ds(ch * C + g * 16, 16)]
            for j in range(16):
                r = g * 16 + j
                buf[r, 0:16] = buf[r, 0:16] * v16[j]
                buf[r, 16:32] = buf[r, 16:32] * v16[j]

    def _block(blk, carry):
        eb = e_base + blk * MB
        pltpu.sync_copy(rows.at[pl.ds(eb, MB)], rvm)
        pltpu.sync_copy(cols.at[pl.ds(eb, MB)], cvm)
        pltpu.sync_copy(vals.at[pl.ds(eb, MB)], vvm)

        def _xform(k, carry2):
            for g in range(NCH * C // 16 // NCH):
                b = k * C + g * 16
                t = cvm[pl.ds(b, 16)]
                ivm[k, pl.ds(g * 16, 16)] = t * 2 + c
                rix[k, pl.ds(g * 16, 16)] = rvm[pl.ds(b, 16)]
            return carry2

        lax.fori_loop(0, NCH, _xform, 0)

        bufs = (gbuf0, gbuf1, gbuf2, gbuf3)
        for b in range(3):
            pltpu.async_copy(x2.at[ivm.at[b]], bufs[b], gsem)

        def _chunk(i, carry2):
            def _process(buf, nxt):
                pltpu.make_async_copy(x2.at[ivm.at[i]], buf, gsem).wait()
                _scale(buf, i)

                @pl.when(i >= 1)
                def _():
                    pltpu.make_async_copy(
                        buf, acc.at[rix.at[0]], ssem).wait()

                @pl.when(i + 3 < NCH)
                def _():
                    pltpu.async_copy(x2.at[ivm.at[i + 3]], nxt, gsem)

                pltpu.async_copy(buf, acc.at[rix.at[i]], ssem, add=True)

            for par in range(4):
                @pl.when(lax.rem(i, 4) == par)
                def _(par=par):
                    _process(bufs[par], bufs[(par + 3) % 4])

            return carry2

        lax.fori_loop(0, NCH, _chunk, 0)
        pltpu.make_async_copy(gbuf0, acc.at[rix.at[0]], ssem).wait()
        return carry

    lax.fori_loop(0, NBLK, _block, 0)
    plsc.subcore_barrier()

    def _wb(k, carry):
        r0 = s * ROWS_PER_TILE + k * WB
        pltpu.sync_copy(acc.at[pl.ds(r0, WB), :], wbuf)
        pltpu.sync_copy(wbuf, out.at[pl.ds(r0, WB), c, :])
        return carry

    lax.fori_loop(0, N // NSUB // WB, _wb, 0)


@jax.jit
def _spmm(x, rows, cols, vals):
    mesh = plsc.VectorSubcoreMesh(core_axis_name="c", subcore_axis_name="s")
    x2 = x.reshape(2 * N, H)
    out = pl.kernel(
        _spmm_body,
        out_type=jax.ShapeDtypeStruct((N, 2, H), _f32),
        mesh=mesh,
        compiler_params=pltpu.CompilerParams(use_tc_tiling_on_sc=False),
        scratch_types=[
            pltpu.VMEM((MB,), jnp.int32),
            pltpu.VMEM((MB,), jnp.int32),
            pltpu.VMEM((MB,), _f32),
            pltpu.VMEM((NCH, C), jnp.int32),
            pltpu.VMEM((NCH, C), jnp.int32),
            pltpu.VMEM((C, H), _f32),
            pltpu.VMEM((C, H), _f32),
            pltpu.VMEM((C, H), _f32),
            pltpu.VMEM((C, H), _f32),
            pltpu.VMEM((WB, H), _f32),
            pltpu.VMEM_SHARED((N, H), _f32),
            pltpu.SemaphoreType.DMA,
            pltpu.SemaphoreType.DMA,
        ],
    )(x2, rows, cols, vals)
    return out.reshape(N, D)


def _gate_k(em_ref, w_ref, b_ref, o0, o1, o2, o3):
    em = em_ref[...]
    for i, o in enumerate((o0, o1, o2, o3)):
        z = jnp.dot(em, w_ref[i], preferred_element_type=_f32) + b_ref[i][None, :]
        o[...] = em * jax.nn.sigmoid(z)


def _attn_mix(u0, u1, u2, am_ref, aa_ref):
    ws = []
    for u in (u0, u1, u2):
        t = jnp.dot(u, am_ref[...], preferred_element_type=_f32)
        ws.append(jnp.sum(aa_ref[...] * t, axis=1))
    m = jnp.maximum(jnp.maximum(ws[0], ws[1]), ws[2])
    es = [jnp.exp(w - m) for w in ws]
    tot = es[0] + es[1] + es[2]
    mixed = es[0][:, None] * u0 + es[1][:, None] * u1 + es[2][:, None] * u2
    return mixed / tot[:, None]


def _mix_k(u0, u1, u2, us, am_ref, aa_ref, mo):
    mixed = _attn_mix(u0[...], u1[...], u2[...], am_ref, aa_ref)
    mo[...] = (mixed + us[...]) * 0.5


def _normacc_k(a0, x0, a1, x1, a2, x2, a3, x3, a4, x4, o0, o1, o2, o3, o4):
    for a, x, o in ((a0, x0, o0), (a1, x1, o1), (a2, x2, o2), (a3, x3, o3),
                    (a4, x4, o4)):
        xx = x[...]
        n = jnp.sqrt(jnp.sum(xx * xx, axis=1, keepdims=True))
        o[...] = a[...] + xx / jnp.maximum(n, 1e-12)


def _final_k(a0, a1, a2, aS, am_ref, aa_ref, sw_ref, sb_ref, fu, s0, s1, s2):
    mixed = _attn_mix(a0[...], a1[...], a2[...], am_ref, aa_ref)
    f = mixed + aS[...] * 0.5
    fu[...] = f
    for i, o in enumerate((s0, s1, s2)):
        z = jnp.dot(f, sw_ref[i], preferred_element_type=_f32) + sb_ref[i][None, :]
        o[...] = f * jax.nn.sigmoid(z)


_row_spec = pl.BlockSpec((RB, D), lambda i: (i, 0))


def _full_spec(shape):
    return pl.BlockSpec(shape, lambda i: tuple(0 for _ in shape))


def _rows_out(n):
    return tuple(jax.ShapeDtypeStruct((N, D), _f32) for _ in range(n))


@jax.jit
def _gate(em, w, b):
    return pl.pallas_call(
        _gate_k,
        grid=(GRID,),
        in_specs=[_row_spec, _full_spec((4, D, D)), _full_spec((4, D))],
        out_specs=(_row_spec,) * 4,
        out_shape=_rows_out(4),
    )(em, w, b)


@jax.jit
def _mix(u0, u1, u2, us, am, aa):
    return pl.pallas_call(
        _mix_k,
        grid=(GRID,),
        in_specs=[_row_spec] * 4 + [_full_spec((D, D)), _full_spec((1, D))],
        out_specs=_row_spec,
        out_shape=jax.ShapeDtypeStruct((N, D), _f32),
    )(u0, u1, u2, us, am, aa)


@jax.jit
def _normacc(a0, x0, a1, x1, a2, x2, a3, x3, a4, x4):
    return pl.pallas_call(
        _normacc_k,
        grid=(GRID,),
        in_specs=[_row_spec] * 10,
        out_specs=(_row_spec,) * 5,
        out_shape=_rows_out(5),
    )(a0, x0, a1, x1, a2, x2, a3, x3, a4, x4)


@jax.jit
def _final(a0, a1, a2, aS, am, aa, sw, sb):
    return pl.pallas_call(
        _final_k,
        grid=(GRID,),
        in_specs=[_row_spec] * 4
        + [_full_spec((D, D)), _full_spec((1, D)), _full_spec((4, D, D)),
           _full_spec((4, D))],
        out_specs=(_row_spec,) * 4,
        out_shape=_rows_out(4),
    )(a0, a1, a2, aS, am, aa, sw, sb)


def kernel(user_emb, item_emb, gating_w, gating_b, sgating_w, sgating_b,
           att_mat, att_agg, hs_rows, hs_cols, hs_vals, hj_rows, hj_cols,
           hj_vals, hp_rows, hp_cols, hp_vals, inter_rows, inter_cols,
           inter_vals):
    i32 = jnp.int32
    hs_rows, hs_cols = hs_rows.astype(i32), hs_cols.astype(i32)
    hj_rows, hj_cols = hj_rows.astype(i32), hj_cols.astype(i32)
    hp_rows, hp_cols = hp_rows.astype(i32), hp_cols.astype(i32)
    inter_rows, inter_cols = inter_rows.astype(i32), inter_cols.astype(i32)

    u0, u1, u2, us = _gate(user_emb, gating_w, gating_b)
    acc0, acc1, acc2, accS, accI = u0, u1, u2, us, item_emb
    it = item_emb
    for _ in range(2):
        mixed = _mix(u0, u1, u2, us, att_mat, att_agg)
        u0n = _spmm(u0, hs_rows, hs_cols, hs_vals)
        u1n = _spmm(u1, hj_rows, hj_cols, hj_vals)
        u2n = _spmm(u2, hp_rows, hp_cols, hp_vals)
        itn = _spmm(mixed, inter_cols, inter_rows, inter_vals)
        usn = _spmm(it, inter_rows, inter_cols, inter_vals)
        acc0, acc1, acc2, accS, accI = _normacc(
            acc0, u0n, acc1, u1n, acc2, u2n, accS, usn, accI, itn)
        u0, u1, u2, us, it = u0n, u1n, u2n, usn, itn
    fu, s0, s1, s2 = _final(acc0, acc1, acc2, accS, att_mat, att_agg,
                            sgating_w, sgating_b)
    return (fu, accI, (s0, s1, s2))
